# baseline (device time: 343867 ns/iter reference)
import functools

import jax
import jax.numpy as jnp
from jax import lax
from jax.experimental import pallas as pl
from jax.experimental.pallas import tpu as pltpu

N_DEV = 8
E_LOC = 8
N_EXP = N_DEV * E_LOC
D = 512
D2 = D // 2
H = 1024
N_STEP = 7


def _plane_moe(x, wg, wz, w_top, w_bot):
    n = x.shape[0]

    def body(x_ref, wg_ref, wz_ref, wt_hbm, wb_hbm, out_ref,
             comm_r, comm_l, z_top, z_bot,
             send_r, recv_r, send_l, recv_l,
             zsend, zrecv, load_sem, credit_r, credit_l):
        s = pl.program_id(0)
        my = lax.axis_index("i")
        p = lax.rem(my, 4)
        base = my - p
        right = base + lax.rem(p + 1, 4)
        left = base + lax.rem(p + 3, 4)
        zn = lax.rem(my + 4, N_DEV)
        cur = lax.rem(s, 2)
        nxt = lax.rem(s + 1, 2)

        zx_t = pltpu.make_async_remote_copy(
            src_ref=wt_hbm, dst_ref=z_top,
            send_sem=zsend.at[0], recv_sem=zrecv.at[0],
            device_id=(zn,), device_id_type=pl.DeviceIdType.MESH)
        zx_b = pltpu.make_async_remote_copy(
            src_ref=wb_hbm, dst_ref=z_bot,
            send_sem=zsend.at[1], recv_sem=zrecv.at[1],
            device_id=(zn,), device_id_type=pl.DeviceIdType.MESH)

        @pl.when(s == 0)
        def _():
            barrier = pltpu.get_barrier_semaphore()
            for nbr in (left, right, zn):
                pl.semaphore_signal(barrier, inc=1, device_id=(nbr,),
                                    device_id_type=pl.DeviceIdType.MESH)
            pl.semaphore_wait(barrier, 3)
            zx_t.start()
            zx_b.start()
            cp_t = pltpu.make_async_copy(wt_hbm, comm_r.at[0], load_sem)
            cp_t.start()
            cp_t.wait()
            cp_b = pltpu.make_async_copy(wb_hbm, comm_l.at[0], load_sem)
            cp_b.start()
            cp_b.wait()
            out_ref[:, :] = jnp.zeros_like(out_ref)

        @pl.when(s == 3)
        def _():
            zx_t.wait()
            zx_b.wait()

        rdma_r = pltpu.make_async_remote_copy(
            src_ref=comm_r.at[cur], dst_ref=comm_r.at[nxt],
            send_sem=send_r.at[cur], recv_sem=recv_r.at[nxt],
            device_id=(right,), device_id_type=pl.DeviceIdType.MESH)
        rdma_l = pltpu.make_async_remote_copy(
            src_ref=comm_l.at[cur], dst_ref=comm_l.at[nxt],
            send_sem=send_l.at[cur], recv_sem=recv_l.at[nxt],
            device_id=(left,), device_id_type=pl.DeviceIdType.MESH)
        zrdma_r = pltpu.make_async_remote_copy(
            src_ref=z_top, dst_ref=comm_r.at[nxt],
            send_sem=send_r.at[cur], recv_sem=recv_r.at[nxt],
            device_id=(right,), device_id_type=pl.DeviceIdType.MESH)
        zrdma_l = pltpu.make_async_remote_copy(
            src_ref=z_bot, dst_ref=comm_l.at[nxt],
            send_sem=send_l.at[cur], recv_sem=recv_l.at[nxt],
            device_id=(left,), device_id_type=pl.DeviceIdType.MESH)

        @pl.when(s < N_STEP - 1)
        def _():
            @pl.when(s >= 1)
            def _():
                pl.semaphore_wait(credit_r, 1)
                pl.semaphore_wait(credit_l, 1)

            @pl.when(s == 3)
            def _():
                zrdma_r.start()
                zrdma_l.start()

            @pl.when(s != 3)
            def _():
                rdma_r.start()
                rdma_l.start()

        xt = x_ref[:, :D2]
        xb = x_ref[:, D2:]

        def dir_dot(xh, w_ref, cols):
            gx = jnp.concatenate(
                [xh * cols[:, j:j + 1].astype(jnp.bfloat16)
                 for j in range(E_LOC)], axis=1)
            wm = w_ref[:, :, :]
            return jnp.dot(gx, wm.reshape(E_LOC * D2, H),
                           preferred_element_type=jnp.float32)

        out_ref[:, :] = out_ref[:, :] + dir_dot(
            xt, comm_r.at[cur], wg_ref[0, :, :E_LOC])
        out_ref[:, :] = out_ref[:, :] + dir_dot(
            xb, comm_l.at[cur], wg_ref[0, :, E_LOC:])

        @pl.when(s == 3)
        def _():
            out_ref[:, :] = out_ref[:, :] + dir_dot(xt, z_top, wz_ref[:, :])
            out_ref[:, :] = out_ref[:, :] + dir_dot(xb, z_bot, wz_ref[:, :])

        @pl.when(s < N_STEP - 1)
        def _():
            rdma_r.wait()
            rdma_l.wait()

            @pl.when(s < N_STEP - 2)
            def _():
                pl.semaphore_signal(credit_r, inc=1, device_id=(left,),
                                    device_id_type=pl.DeviceIdType.MESH)
                pl.semaphore_signal(credit_l, inc=1, device_id=(right,),
                                    device_id_type=pl.DeviceIdType.MESH)

        @pl.when(s == N_STEP - 1)
        def _():
            @functools.partial(pl.run_scoped,
                               exit_sem=pltpu.SemaphoreType.REGULAR)
            def _(exit_sem):
                for nbr in (left, right, zn):
                    pl.semaphore_signal(exit_sem, inc=1, device_id=(nbr,),
                                        device_id_type=pl.DeviceIdType.MESH)
                pl.semaphore_wait(exit_sem, 3)

    return pl.pallas_call(
        body,
        grid=(N_STEP,),
        out_shape=jax.ShapeDtypeStruct((n, H), jnp.float32),
        in_specs=[
            pl.BlockSpec((n, D), lambda s: (0, 0)),
            pl.BlockSpec((1, n, 2 * E_LOC), lambda s: (s, 0, 0)),
            pl.BlockSpec((n, E_LOC), lambda s: (0, 0)),
            pl.BlockSpec(memory_space=pl.ANY),
            pl.BlockSpec(memory_space=pl.ANY),
        ],
        out_specs=pl.BlockSpec((n, H), lambda s: (0, 0)),
        scratch_shapes=[
            pltpu.VMEM((2, E_LOC, D2, H), jnp.bfloat16),
            pltpu.VMEM((2, E_LOC, D2, H), jnp.bfloat16),
            pltpu.VMEM((E_LOC, D2, H), jnp.bfloat16),
            pltpu.VMEM((E_LOC, D2, H), jnp.bfloat16),
            pltpu.SemaphoreType.DMA((2,)),
            pltpu.SemaphoreType.DMA((2,)),
            pltpu.SemaphoreType.DMA((2,)),
            pltpu.SemaphoreType.DMA((2,)),
            pltpu.SemaphoreType.DMA((2,)),
            pltpu.SemaphoreType.DMA((2,)),
            pltpu.SemaphoreType.DMA,
            pltpu.SemaphoreType.REGULAR,
            pltpu.SemaphoreType.REGULAR,
        ],
        compiler_params=pltpu.CompilerParams(
            collective_id=0,
            vmem_limit_bytes=62 * 1024 * 1024,
            dimension_semantics=("arbitrary",),
        ),
    )(x, wg, wz, w_top, w_bot)


def kernel(x, router_W, route_idx, expert_W):
    n = x.shape[0]

    scores = x @ router_W
    probs = jax.nn.softmax(scores, axis=-1)

    eids = jnp.arange(N_EXP, dtype=route_idx.dtype)[None, :]
    oh0 = route_idx[:, 0:1] == eids
    oh1 = route_idx[:, 1:2] == eids
    g0 = jnp.sum(jnp.where(oh0, probs, 0.0), axis=-1, keepdims=True)
    g1 = jnp.sum(jnp.where(oh1, probs, 0.0), axis=-1, keepdims=True)
    gs = g0 + g1
    w_dense = jnp.where(oh0, g0 / gs, 0.0) + jnp.where(oh1, g1 / gs, 0.0)

    my = lax.axis_index("i")
    p = my % 4
    base = my - p
    s_arr = jnp.arange(N_STEP)
    hop = jnp.where(s_arr <= 3, s_arr, s_arr - 3)
    plane_off = jnp.where(s_arr <= 3, 0, 4)
    idx_r = (base + (p - hop) % 4 + plane_off) % N_DEV
    idx_l = (base + (p + hop) % 4 + plane_off) % N_DEV

    blocks = w_dense.reshape(n, N_DEV, E_LOC)
    wg_r = jnp.take(blocks, idx_r, axis=1)
    wg_l = jnp.take(blocks, idx_l, axis=1)
    wg = jnp.concatenate([wg_r, wg_l], axis=2)
    wg = jnp.transpose(wg, (1, 0, 2))
    wz = blocks[:, (my + 4) % N_DEV, :]

    w_bf = expert_W.astype(jnp.bfloat16)
    return _plane_moe(x.astype(jnp.bfloat16), wg, wz,
                      w_bf[:, :D2, :], w_bf[:, D2:, :])
